# baseline (speedup 1.0000x reference)
import functools

import jax
import jax.numpy as jnp
from jax import lax
from jax.experimental import pallas as pl
from jax.experimental.pallas import tpu as pltpu
from jax.experimental.pallas import tpu_sc as plsc

E_BLOCK = 3200
NUM_SPECIES = 16
NUM_SCALAR = 64
L = 16
K_LHS = 112
N_OUT = 368


def _r0_edge_sparsecore(edge_index, atom_type, r0_pad16):
    E = edge_index.shape[1]
    N = atom_type.shape[0]
    info = plsc.get_sparse_core_info()
    NC, NS = info.num_cores, info.num_subcores
    NW = NC * NS
    per_w = E // NW
    n_grp = per_w // L
    tail = per_w - n_grp * L

    mesh = plsc.VectorSubcoreMesh(core_axis_name="c", subcore_axis_name="s")

    @functools.partial(
        pl.kernel,
        mesh=mesh,
        out_type=jax.ShapeDtypeStruct((E,), jnp.float32),
        scratch_types=[
            pltpu.VMEM((N,), jnp.int32),
            pltpu.VMEM((per_w,), jnp.int32),
            pltpu.VMEM((per_w,), jnp.int32),
            pltpu.VMEM((per_w,), jnp.float32),
            pltpu.VMEM((L,), jnp.float32),
            pltpu.SemaphoreType.DMA,
        ],
        compiler_params=pltpu.CompilerParams(needs_layout_passes=False),
    )
    def k(ei_hbm, at_hbm, r0_hbm, out_hbm, at_v, i0_v, i1_v, re_v, r0_v, sem):
        wid = lax.axis_index("s") * NC + lax.axis_index("c")
        base = wid * per_w
        cps = [
            pltpu.async_copy(at_hbm, at_v, sem),
            pltpu.async_copy(r0_hbm, r0_v, sem),
            pltpu.async_copy(ei_hbm.at[pl.ds(base, per_w)], i0_v, sem),
            pltpu.async_copy(ei_hbm.at[pl.ds(E + base, per_w)], i1_v, sem),
        ]
        for cp in cps:
            cp.wait()

        def group(off):
            ta = plsc.load_gather(at_v, [i0_v[pl.ds(off, L)]])
            tb = plsc.load_gather(at_v, [i1_v[pl.ds(off, L)]])
            ra = plsc.load_gather(r0_v, [ta])
            rb = plsc.load_gather(r0_v, [tb])
            re_v[pl.ds(off, L)] = 0.5 * (ra + rb)

        def body(g, carry):
            group(g * L)
            return carry

        lax.fori_loop(0, n_grp, body, 0)
        if tail:
            group(per_w - L)

        pltpu.sync_copy(re_v, out_hbm.at[pl.ds(base, per_w)])

    return k(edge_index.reshape(-1), atom_type, r0_pad16)


def _tc_body(aux_ref, x_ref, t_ref, o_ref):
    aux = aux_ref[0]
    et = aux[0:1]
    r = aux[1:2]
    rr = aux[2:3]
    oh = (lax.broadcasted_iota(jnp.int32, (NUM_SPECIES, E_BLOCK), 0)
          == et.astype(jnp.int32)).astype(jnp.float32)
    lg = jnp.log(r / rr)
    rows = [oh]
    rj = r
    for _ in range(5):
        rows.append(rj * oh)
        rj = rj * r
    rows.append(lg * oh)
    lhs = jnp.concatenate(rows, axis=0)
    m = lax.dot_general(lhs, t_ref[...], (((0,), (0,)), ((), ())),
                        preferred_element_type=jnp.float32)
    x = x_ref[...]
    scv = m[:, :240]
    p = m[:, 240:304]
    z = m[:, 304:368]
    sh = p * jnp.exp(z)
    scaled = scv * x
    o_ref[:, :NUM_SCALAR] = scaled[:, :NUM_SCALAR] + sh
    o_ref[:, NUM_SCALAR:] = scaled[:, NUM_SCALAR:]


def kernel(in_field, edge_index, edge_type, atom_type, edge_length, scales, shifts, r0):
    E, D = in_field.shape
    G = E // E_BLOCK

    r0_pad16 = jnp.pad(r0, (0, L - r0.shape[0]))
    r0_edge = _r0_edge_sparsecore(edge_index, atom_type, r0_pad16)

    scales_exp = jnp.concatenate(
        [scales[:, :NUM_SCALAR],
         jnp.repeat(scales[:, 64:96], 3, axis=1),
         jnp.repeat(scales[:, 96:112], 5, axis=1)], axis=1)
    tbl = jnp.zeros((K_LHS, N_OUT), jnp.float32)
    tbl = tbl.at[0:16, 0:240].set(scales_exp)
    for j in range(6):
        tbl = tbl.at[16 * j:16 * (j + 1), 240:304].set(shifts[:, :, j])
    tbl = tbl.at[96:112, 304:368].set(-(1.0 + jnp.abs(shifts[:, :, 6])))

    aux = jnp.zeros((G, 8, E_BLOCK), jnp.float32)
    aux = aux.at[:, 0, :].set(edge_type.reshape(G, E_BLOCK).astype(jnp.float32))
    aux = aux.at[:, 1, :].set(edge_length.reshape(G, E_BLOCK))
    aux = aux.at[:, 2, :].set(r0_edge.reshape(G, E_BLOCK))

    return pl.pallas_call(
        _tc_body,
        grid=(G,),
        in_specs=[
            pl.BlockSpec((1, 8, E_BLOCK), lambda i: (i, 0, 0)),
            pl.BlockSpec((E_BLOCK, D), lambda i: (i, 0)),
            pl.BlockSpec((K_LHS, N_OUT), lambda i: (0, 0)),
        ],
        out_specs=pl.BlockSpec((E_BLOCK, D), lambda i: (i, 0)),
        out_shape=jax.ShapeDtypeStruct((E, D), jnp.float32),
        compiler_params=pltpu.CompilerParams(
            dimension_semantics=("parallel",),
        ),
    )(aux, in_field, tbl)


# speedup vs baseline: 1.0208x; 1.0208x over previous
import functools

import jax
import jax.numpy as jnp
from jax import lax
from jax.experimental import pallas as pl
from jax.experimental.pallas import tpu as pltpu
from jax.experimental.pallas import tpu_sc as plsc

E_BLOCK = 6400
NUM_SPECIES = 16
NUM_SCALAR = 64
L = 16
K_LHS = 112
N_OUT = 368


def _r0_edge_sparsecore(edge_index, atom_type, r0_pad16):
    E = edge_index.shape[1]
    N = atom_type.shape[0]
    info = plsc.get_sparse_core_info()
    NC, NS = info.num_cores, info.num_subcores
    NW = NC * NS
    per_w = E // NW
    n_grp = per_w // L
    tail = per_w - n_grp * L

    mesh = plsc.VectorSubcoreMesh(core_axis_name="c", subcore_axis_name="s")

    @functools.partial(
        pl.kernel,
        mesh=mesh,
        out_type=jax.ShapeDtypeStruct((E,), jnp.float32),
        scratch_types=[
            pltpu.VMEM((N,), jnp.int32),
            pltpu.VMEM((per_w,), jnp.int32),
            pltpu.VMEM((per_w,), jnp.int32),
            pltpu.VMEM((per_w,), jnp.float32),
            pltpu.VMEM((L,), jnp.float32),
            pltpu.SemaphoreType.DMA,
        ],
        compiler_params=pltpu.CompilerParams(needs_layout_passes=False),
    )
    def k(ei_hbm, at_hbm, r0_hbm, out_hbm, at_v, i0_v, i1_v, re_v, r0_v, sem):
        wid = lax.axis_index("s") * NC + lax.axis_index("c")
        base = wid * per_w
        cps = [
            pltpu.async_copy(at_hbm, at_v, sem),
            pltpu.async_copy(r0_hbm, r0_v, sem),
            pltpu.async_copy(ei_hbm.at[pl.ds(base, per_w)], i0_v, sem),
            pltpu.async_copy(ei_hbm.at[pl.ds(E + base, per_w)], i1_v, sem),
        ]
        for cp in cps:
            cp.wait()

        def group(off):
            ta = plsc.load_gather(at_v, [i0_v[pl.ds(off, L)]])
            tb = plsc.load_gather(at_v, [i1_v[pl.ds(off, L)]])
            ra = plsc.load_gather(r0_v, [ta])
            rb = plsc.load_gather(r0_v, [tb])
            re_v[pl.ds(off, L)] = 0.5 * (ra + rb)

        def body(g, carry):
            group(g * L)
            return carry

        lax.fori_loop(0, n_grp, body, 0)
        if tail:
            group(per_w - L)

        pltpu.sync_copy(re_v, out_hbm.at[pl.ds(base, per_w)])

    return k(edge_index.reshape(-1), atom_type, r0_pad16)


def _tc_body(aux_ref, x_ref, t_ref, o_ref):
    aux = aux_ref[0]
    et = aux[0:1]
    r = aux[1:2]
    rr = aux[2:3]
    oh = (lax.broadcasted_iota(jnp.int32, (NUM_SPECIES, E_BLOCK), 0)
          == et.astype(jnp.int32)).astype(jnp.float32)
    lg = jnp.log(r / rr)
    rows = [oh]
    rj = r
    for _ in range(5):
        rows.append(rj * oh)
        rj = rj * r
    rows.append(lg * oh)
    lhs = jnp.concatenate(rows, axis=0)
    m = lax.dot_general(lhs, t_ref[...], (((0,), (0,)), ((), ())),
                        preferred_element_type=jnp.float32)
    x = x_ref[...]
    scv = m[:, :240]
    p = m[:, 240:304]
    z = m[:, 304:368]
    sh = p * jnp.exp(z)
    scaled = scv * x
    o_ref[:, :NUM_SCALAR] = scaled[:, :NUM_SCALAR] + sh
    o_ref[:, NUM_SCALAR:] = scaled[:, NUM_SCALAR:]


def kernel(in_field, edge_index, edge_type, atom_type, edge_length, scales, shifts, r0):
    E, D = in_field.shape
    G = E // E_BLOCK

    r0_pad16 = jnp.pad(r0, (0, L - r0.shape[0]))
    r0_edge = _r0_edge_sparsecore(edge_index, atom_type, r0_pad16)

    scales_exp = jnp.concatenate(
        [scales[:, :NUM_SCALAR],
         jnp.repeat(scales[:, 64:96], 3, axis=1),
         jnp.repeat(scales[:, 96:112], 5, axis=1)], axis=1)
    tbl = jnp.zeros((K_LHS, N_OUT), jnp.float32)
    tbl = tbl.at[0:16, 0:240].set(scales_exp)
    for j in range(6):
        tbl = tbl.at[16 * j:16 * (j + 1), 240:304].set(shifts[:, :, j])
    tbl = tbl.at[96:112, 304:368].set(-(1.0 + jnp.abs(shifts[:, :, 6])))

    aux = jnp.zeros((G, 8, E_BLOCK), jnp.float32)
    aux = aux.at[:, 0, :].set(edge_type.reshape(G, E_BLOCK).astype(jnp.float32))
    aux = aux.at[:, 1, :].set(edge_length.reshape(G, E_BLOCK))
    aux = aux.at[:, 2, :].set(r0_edge.reshape(G, E_BLOCK))

    return pl.pallas_call(
        _tc_body,
        grid=(G,),
        in_specs=[
            pl.BlockSpec((1, 8, E_BLOCK), lambda i: (i, 0, 0)),
            pl.BlockSpec((E_BLOCK, D), lambda i: (i, 0)),
            pl.BlockSpec((K_LHS, N_OUT), lambda i: (0, 0)),
        ],
        out_specs=pl.BlockSpec((E_BLOCK, D), lambda i: (i, 0)),
        out_shape=jax.ShapeDtypeStruct((E, D), jnp.float32),
        compiler_params=pltpu.CompilerParams(
            dimension_semantics=("parallel",),
        ),
    )(aux, in_field, tbl)


# speedup vs baseline: 1.0547x; 1.0331x over previous
import functools

import jax
import jax.numpy as jnp
from jax import lax
from jax.experimental import pallas as pl
from jax.experimental.pallas import tpu as pltpu
from jax.experimental.pallas import tpu_sc as plsc

E_BLOCK = 6400
NUM_SPECIES = 16
NUM_SCALAR = 64
L = 16
K_LHS = 112
N_OUT = 368


def _r0_edge_sparsecore(edge_index, atom_type, r0_pad16):
    E = edge_index.shape[1]
    N = atom_type.shape[0]
    info = plsc.get_sparse_core_info()
    NC, NS = info.num_cores, info.num_subcores
    NW = NC * NS
    per_w = E // NW
    n_grp = per_w // L
    tail = per_w - n_grp * L

    mesh = plsc.VectorSubcoreMesh(core_axis_name="c", subcore_axis_name="s")

    @functools.partial(
        pl.kernel,
        mesh=mesh,
        out_type=jax.ShapeDtypeStruct((E,), jnp.float32),
        scratch_types=[
            pltpu.VMEM((N,), jnp.int32),
            pltpu.VMEM((per_w,), jnp.int32),
            pltpu.VMEM((per_w,), jnp.int32),
            pltpu.VMEM((per_w,), jnp.float32),
            pltpu.VMEM((L,), jnp.float32),
            pltpu.SemaphoreType.DMA,
        ],
        compiler_params=pltpu.CompilerParams(needs_layout_passes=False),
    )
    def k(ei_hbm, at_hbm, r0_hbm, out_hbm, at_v, i0_v, i1_v, re_v, r0_v, sem):
        wid = lax.axis_index("s") * NC + lax.axis_index("c")
        base = wid * per_w
        cps = [
            pltpu.async_copy(at_hbm, at_v, sem),
            pltpu.async_copy(r0_hbm, r0_v, sem),
            pltpu.async_copy(ei_hbm.at[pl.ds(base, per_w)], i0_v, sem),
            pltpu.async_copy(ei_hbm.at[pl.ds(E + base, per_w)], i1_v, sem),
        ]
        for cp in cps:
            cp.wait()

        def group(off):
            ta = plsc.load_gather(at_v, [i0_v[pl.ds(off, L)]])
            tb = plsc.load_gather(at_v, [i1_v[pl.ds(off, L)]])
            ra = plsc.load_gather(r0_v, [ta])
            rb = plsc.load_gather(r0_v, [tb])
            re_v[pl.ds(off, L)] = 0.5 * (ra + rb)

        def body(g, carry):
            group(g * L)
            return carry

        lax.fori_loop(0, n_grp, body, 0)
        if tail:
            group(per_w - L)

        pltpu.sync_copy(re_v, out_hbm.at[pl.ds(base, per_w)])

    return k(edge_index.reshape(-1), atom_type, r0_pad16)


def _tc_body(aux_ref, x_ref, t_ref, o_ref):
    aux = aux_ref[0]
    et = aux[0:1]
    r = aux[1:2]
    rr = aux[2:3]
    oh = (lax.broadcasted_iota(jnp.int32, (NUM_SPECIES, E_BLOCK), 0)
          == et.astype(jnp.int32)).astype(jnp.float32)
    lg = jnp.log(r / rr)
    rows = [oh]
    rj = r
    for _ in range(5):
        rows.append(rj * oh)
        rj = rj * r
    rows.append(lg * oh)
    lhs = jnp.concatenate(rows, axis=0)
    m = lax.dot_general(lhs, t_ref[...], (((0,), (0,)), ((), ())),
                        preferred_element_type=jnp.float32)
    x = x_ref[...]
    p = m[:, 0:64]
    z = m[:, 64:128]
    scv = m[:, 128:368]
    sh = p * jnp.exp(z)
    scaled = scv * x
    o_ref[:, :NUM_SCALAR] = scaled[:, :NUM_SCALAR] + sh
    o_ref[:, NUM_SCALAR:] = scaled[:, NUM_SCALAR:]


def kernel(in_field, edge_index, edge_type, atom_type, edge_length, scales, shifts, r0):
    E, D = in_field.shape
    G = E // E_BLOCK

    r0_pad16 = jnp.pad(r0, (0, L - r0.shape[0]))
    r0_edge = _r0_edge_sparsecore(edge_index, atom_type, r0_pad16)

    scales_exp = jnp.concatenate(
        [scales[:, :NUM_SCALAR],
         jnp.repeat(scales[:, 64:96], 3, axis=1),
         jnp.repeat(scales[:, 96:112], 5, axis=1)], axis=1)
    tbl = jnp.zeros((K_LHS, N_OUT), jnp.float32)
    tbl = tbl.at[0:16, 128:368].set(scales_exp)
    for j in range(6):
        tbl = tbl.at[16 * j:16 * (j + 1), 0:64].set(shifts[:, :, j])
    tbl = tbl.at[96:112, 64:128].set(-(1.0 + jnp.abs(shifts[:, :, 6])))

    aux = jnp.zeros((G, 8, E_BLOCK), jnp.float32)
    aux = aux.at[:, 0, :].set(edge_type.reshape(G, E_BLOCK).astype(jnp.float32))
    aux = aux.at[:, 1, :].set(edge_length.reshape(G, E_BLOCK))
    aux = aux.at[:, 2, :].set(r0_edge.reshape(G, E_BLOCK))

    return pl.pallas_call(
        _tc_body,
        grid=(G,),
        in_specs=[
            pl.BlockSpec((1, 8, E_BLOCK), lambda i: (i, 0, 0)),
            pl.BlockSpec((E_BLOCK, D), lambda i: (i, 0)),
            pl.BlockSpec((K_LHS, N_OUT), lambda i: (0, 0)),
        ],
        out_specs=pl.BlockSpec((E_BLOCK, D), lambda i: (i, 0)),
        out_shape=jax.ShapeDtypeStruct((E, D), jnp.float32),
        compiler_params=pltpu.CompilerParams(
            dimension_semantics=("parallel",),
        ),
    )(aux, in_field, tbl)
